# initial kernel scaffold (unmeasured)
import jax
import jax.numpy as jnp
from jax import lax
from jax.experimental import pallas as pl
from jax.experimental.pallas import tpu as pltpu

N_DEV = 8
M = 8192
N = 4096
CHUNK = M // N_DEV


def kernel(x, w_mat):
    partial = jnp.dot(x, w_mat, preferred_element_type=jnp.float32)

    def body(p_ref, out_ref, acc, tmp, send_sems, recv_sems, load_sem,
             store_sem, credit_sem):
        my = lax.axis_index("i")
        left = lax.rem(my - 1 + N_DEV, N_DEV)
        right = lax.rem(my + 1, N_DEV)

        barrier = pltpu.get_barrier_semaphore()
        for nbr in (left, right):
            pl.semaphore_signal(
                barrier, inc=1, device_id=(nbr,),
                device_id_type=pl.DeviceIdType.MESH,
            )
        pl.semaphore_wait(barrier, 2)

        def chunk_rows(c):
            return pl.ds(c * CHUNK, CHUNK)

        cp = pltpu.make_async_copy(
            p_ref.at[chunk_rows(my), :], acc.at[0], load_sem)
        cp.start()
        cp.wait()

        for s in range(N_DEV - 1):
            send_slot = s % 2
            recv_slot = (s + 1) % 2
            if s >= 1:
                pl.semaphore_wait(credit_sem, 1)
            rdma = pltpu.make_async_remote_copy(
                src_ref=acc.at[send_slot],
                dst_ref=acc.at[recv_slot],
                send_sem=send_sems.at[send_slot],
                recv_sem=recv_sems.at[recv_slot],
                device_id=(right,),
                device_id_type=pl.DeviceIdType.MESH,
            )
            rdma.start()
            c = lax.rem(my - s - 1 + N_DEV, N_DEV)
            ld = pltpu.make_async_copy(
                p_ref.at[chunk_rows(c), :], tmp, load_sem)
            ld.start()
            rdma.wait()
            pl.semaphore_signal(
                credit_sem, inc=1, device_id=(left,),
                device_id_type=pl.DeviceIdType.MESH,
            )
            ld.wait()
            acc[recv_slot] = acc[recv_slot] + tmp[...]

        own = lax.rem(my + 1, N_DEV)
        acc[1] = jnp.maximum(acc[1], 0.0)
        st = pltpu.make_async_copy(
            acc.at[1], out_ref.at[chunk_rows(own), :], store_sem)
        st.start()
        st.wait()

        for t in range(N_DEV - 1):
            s = N_DEV - 1 + t
            send_slot = s % 2
            recv_slot = (s + 1) % 2
            pl.semaphore_wait(credit_sem, 1)
            rdma = pltpu.make_async_remote_copy(
                src_ref=acc.at[send_slot],
                dst_ref=acc.at[recv_slot],
                send_sem=send_sems.at[send_slot],
                recv_sem=recv_sems.at[recv_slot],
                device_id=(right,),
                device_id_type=pl.DeviceIdType.MESH,
            )
            rdma.start()
            rdma.wait()
            if s <= 2 * (N_DEV - 1) - 2:
                pl.semaphore_signal(
                    credit_sem, inc=1, device_id=(left,),
                    device_id_type=pl.DeviceIdType.MESH,
                )
            c = lax.rem(my - t + N_DEV, N_DEV)
            st = pltpu.make_async_copy(
                acc.at[recv_slot], out_ref.at[chunk_rows(c), :], store_sem)
            st.start()
            st.wait()

    return pl.pallas_call(
        body,
        out_shape=jax.ShapeDtypeStruct((M, N), jnp.float32),
        in_specs=[pl.BlockSpec(memory_space=pltpu.ANY)],
        out_specs=pl.BlockSpec(memory_space=pltpu.ANY),
        scratch_shapes=[
            pltpu.VMEM((2, CHUNK, N), jnp.float32),
            pltpu.VMEM((CHUNK, N), jnp.float32),
            pltpu.SemaphoreType.DMA((2,)),
            pltpu.SemaphoreType.DMA((2,)),
            pltpu.SemaphoreType.DMA,
            pltpu.SemaphoreType.DMA,
            pltpu.SemaphoreType.REGULAR,
        ],
        compiler_params=pltpu.CompilerParams(collective_id=0),
    )(partial)


# baseline (device time: 1510527 ns/iter reference)
import jax
import jax.numpy as jnp
from jax import lax
from jax.experimental import pallas as pl
from jax.experimental.pallas import tpu as pltpu

N_DEV = 8
N_STEP = 2 * (N_DEV - 1)


class _Ring:
    def __init__(self, idx, acc, snd, rcv, fwd, q):
        self.idx, self.acc, self.snd, self.rcv, self.fwd, self.q = (
            idx, acc, snd, rcv, fwd, q)


def kernel(x, w_mat):
    M = x.shape[0]
    N = w_mat.shape[1]
    CHUNK = M // N_DEV
    Q = N // 4

    partial = jnp.dot(x, w_mat, preferred_element_type=jnp.float32)

    def body(p_ref, out_ref,
             acc0, acc1, acc2, acc3, tmp_f, tmp_r,
             snd0, rcv0, snd1, rcv1, snd2, rcv2, snd3, rcv3,
             ld_f_sem, ld_r_sem, store_sems,
             cr0, cr1, cr2, cr3):
        my = lax.axis_index("i")
        left = lax.rem(my - 1 + N_DEV, N_DEV)
        right = lax.rem(my + 1, N_DEV)

        barrier = pltpu.get_barrier_semaphore()
        for nbr in (left, right):
            pl.semaphore_signal(
                barrier, inc=1, device_id=(nbr,),
                device_id_type=pl.DeviceIdType.MESH,
            )
        pl.semaphore_wait(barrier, 2)

        rings = [
            _Ring(0, acc0, snd0, rcv0, True, 0),
            _Ring(1, acc1, snd1, rcv1, True, 1),
            _Ring(2, acc2, snd2, rcv2, False, 2),
            _Ring(3, acc3, snd3, rcv3, False, 3),
        ]
        credits = [cr0, cr1, cr2, cr3]
        for r in rings:
            r.dst = right if r.fwd else left
            r.credit_to = left if r.fwd else right
            r.tmp = tmp_f if r.fwd else tmp_r
            r.ld_sem = ld_f_sem if r.fwd else ld_r_sem
            r.credit = credits[r.idx]
        pairs = [(rings[0], rings[2]), (rings[1], rings[3])]

        def rows(c):
            return pl.ds(c * CHUNK, CHUNK)

        def cols(q):
            return pl.ds(q * Q, Q)

        def mod8(v):
            return lax.rem(v + 2 * N_DEV, N_DEV)

        def add_chunk_idx(r, s):
            return mod8(my - s - 1) if r.fwd else mod8(my + s + 1)

        def own_chunk_idx(r):
            return mod8(my + 1) if r.fwd else mod8(my - 1)

        def ag_chunk_idx(r, t):
            return mod8(my - t) if r.fwd else mod8(my + t)

        def add_tmp(r, slot):
            for kk in range(2):
                sub = pl.ds(kk * (CHUNK // 2), CHUNK // 2)
                r.acc[slot, sub] = r.acc[slot, sub] + r.tmp[sub]

        def relu_slot(r, slot):
            for kk in range(2):
                sub = pl.ds(kk * (CHUNK // 2), CHUNK // 2)
                r.acc[slot, sub] = jnp.maximum(r.acc[slot, sub], 0.0)

        def start_store(r, slot, c):
            d = pltpu.make_async_copy(
                r.acc.at[slot], out_ref.at[rows(c), cols(r.q)],
                store_sems.at[r.idx])
            d.start()
            return d

        def start_load(r, c):
            d = pltpu.make_async_copy(
                p_ref.at[rows(c), cols(r.q)], r.tmp, r.ld_sem)
            d.start()
            return d

        seeds = []
        for r in rings:
            d = pltpu.make_async_copy(
                p_ref.at[rows(my), cols(r.q)], r.acc.at[0],
                store_sems.at[r.idx])
            d.start()
            seeds.append(d)
        for d in seeds:
            d.wait()

        pending_store = [None, None, None, None]
        prev_ld = [None, None]

        for k in range(2 * N_STEP):
            p = k % 2
            s = k // 2
            pair = pairs[p]

            if s >= 1:
                for r in pair:
                    pl.semaphore_wait(r.credit, 1)

            rdmas = []
            for r in pair:
                rd = pltpu.make_async_remote_copy(
                    src_ref=r.acc.at[s % 2],
                    dst_ref=r.acc.at[(s + 1) % 2],
                    send_sem=r.snd.at[s % 2],
                    recv_sem=r.rcv.at[(s + 1) % 2],
                    device_id=(r.dst,),
                    device_id_type=pl.DeviceIdType.MESH,
                )
                rd.start()
                rdmas.append(rd)

            if k >= 1:
                sp = (k - 1) // 2
                if sp <= N_DEV - 2:
                    for i, r in enumerate(pairs[1 - p]):
                        prev_ld[i].wait()
                        add_tmp(r, (sp + 1) % 2)
                    if sp == N_DEV - 2:
                        for r in pairs[1 - p]:
                            relu_slot(r, 1)
                            pending_store[r.idx] = start_store(
                                r, 1, own_chunk_idx(r))

            if s <= N_DEV - 2:
                for i, r in enumerate(pair):
                    prev_ld[i] = start_load(r, add_chunk_idx(r, s))

            for rd in rdmas:
                rd.wait()
            for r in pair:
                if pending_store[r.idx] is not None:
                    pending_store[r.idx].wait()
                    pending_store[r.idx] = None
                if s <= N_STEP - 2:
                    pl.semaphore_signal(
                        r.credit, inc=1, device_id=(r.credit_to,),
                        device_id_type=pl.DeviceIdType.MESH,
                    )

            if s >= N_DEV - 1:
                t = s - (N_DEV - 1)
                for r in pair:
                    pending_store[r.idx] = start_store(
                        r, (s + 1) % 2, ag_chunk_idx(r, t))

        for d in pending_store:
            if d is not None:
                d.wait()

    return pl.pallas_call(
        body,
        out_shape=jax.ShapeDtypeStruct((M, N), jnp.float32),
        in_specs=[pl.BlockSpec(memory_space=pl.ANY)],
        out_specs=pl.BlockSpec(memory_space=pl.ANY),
        scratch_shapes=[
            pltpu.VMEM((2, CHUNK, Q), jnp.float32),
            pltpu.VMEM((2, CHUNK, Q), jnp.float32),
            pltpu.VMEM((2, CHUNK, Q), jnp.float32),
            pltpu.VMEM((2, CHUNK, Q), jnp.float32),
            pltpu.VMEM((CHUNK, Q), jnp.float32),
            pltpu.VMEM((CHUNK, Q), jnp.float32),
            pltpu.SemaphoreType.DMA((2,)),
            pltpu.SemaphoreType.DMA((2,)),
            pltpu.SemaphoreType.DMA((2,)),
            pltpu.SemaphoreType.DMA((2,)),
            pltpu.SemaphoreType.DMA((2,)),
            pltpu.SemaphoreType.DMA((2,)),
            pltpu.SemaphoreType.DMA((2,)),
            pltpu.SemaphoreType.DMA((2,)),
            pltpu.SemaphoreType.DMA,
            pltpu.SemaphoreType.DMA,
            pltpu.SemaphoreType.DMA((4,)),
            pltpu.SemaphoreType.REGULAR,
            pltpu.SemaphoreType.REGULAR,
            pltpu.SemaphoreType.REGULAR,
            pltpu.SemaphoreType.REGULAR,
        ],
        compiler_params=pltpu.CompilerParams(
            collective_id=0,
            vmem_limit_bytes=60 * 1024 * 1024,
        ),
    )(partial)


# device time: 1453159 ns/iter; 1.0395x vs baseline; 1.0395x over previous
import jax
import jax.numpy as jnp
from jax import lax
from jax.experimental import pallas as pl
from jax.experimental.pallas import tpu as pltpu

N_DEV = 8
N_STEP = 2 * (N_DEV - 1)


class _Ring:
    def __init__(self, idx, acc, snd, rcv, fwd, q):
        self.idx, self.acc, self.snd, self.rcv, self.fwd, self.q = (
            idx, acc, snd, rcv, fwd, q)


def kernel(x, w_mat):
    M = x.shape[0]
    N = w_mat.shape[1]
    CHUNK = M // N_DEV
    Q = N // 4

    partial = jnp.dot(x, w_mat, preferred_element_type=jnp.float32)

    def body(p_ref, out_ref,
             acc0, acc1, acc2, acc3, tmp_f, tmp_r,
             snd0, rcv0, snd1, rcv1, snd2, rcv2, snd3, rcv3,
             ld_f_sem, ld_r_sem, store_sems,
             cr0, cr1, cr2, cr3):
        my = lax.axis_index("i")
        left = lax.rem(my - 1 + N_DEV, N_DEV)
        right = lax.rem(my + 1, N_DEV)

        barrier = pltpu.get_barrier_semaphore()
        for nbr in (left, right):
            pl.semaphore_signal(
                barrier, inc=1, device_id=(nbr,),
                device_id_type=pl.DeviceIdType.MESH,
            )
        pl.semaphore_wait(barrier, 2)

        rings = [
            _Ring(0, acc0, snd0, rcv0, True, 0),
            _Ring(1, acc1, snd1, rcv1, True, 1),
            _Ring(2, acc2, snd2, rcv2, False, 2),
            _Ring(3, acc3, snd3, rcv3, False, 3),
        ]
        credits = [cr0, cr1, cr2, cr3]
        for r in rings:
            r.dst = right if r.fwd else left
            r.credit_to = left if r.fwd else right
            r.tmp = tmp_f if r.fwd else tmp_r
            r.ld_sem = ld_f_sem if r.fwd else ld_r_sem
            r.credit = credits[r.idx]
        pairs = [(rings[0], rings[2]), (rings[1], rings[3])]

        def rows(c):
            return pl.ds(c * CHUNK, CHUNK)

        def cols(q):
            return pl.ds(q * Q, Q)

        def mod8(v):
            return lax.rem(v + 2 * N_DEV, N_DEV)

        def add_chunk_idx(r, s):
            return mod8(my - s - 1) if r.fwd else mod8(my + s + 1)

        def own_chunk_idx(r):
            return mod8(my + 1) if r.fwd else mod8(my - 1)

        def ag_chunk_idx(r, t):
            return mod8(my - t) if r.fwd else mod8(my + t)

        def add_tmp(r, slot):
            for kk in range(2):
                sub = pl.ds(kk * (CHUNK // 2), CHUNK // 2)
                r.acc[slot, sub] = r.acc[slot, sub] + r.tmp[sub]

        def relu_slot(r, slot):
            for kk in range(2):
                sub = pl.ds(kk * (CHUNK // 2), CHUNK // 2)
                r.acc[slot, sub] = jnp.maximum(r.acc[slot, sub], 0.0)

        def start_store(r, slot, c):
            d = pltpu.make_async_copy(
                r.acc.at[slot], out_ref.at[rows(c), cols(r.q)],
                store_sems.at[r.idx])
            d.start()
            return d

        def start_load(r, c):
            d = pltpu.make_async_copy(
                p_ref.at[rows(c), cols(r.q)], r.tmp, r.ld_sem)
            d.start()
            return d

        seeds = []
        for r in rings:
            d = pltpu.make_async_copy(
                p_ref.at[rows(my), cols(r.q)], r.acc.at[0],
                store_sems.at[r.idx])
            d.start()
            seeds.append(d)
        for d in seeds:
            d.wait()

        pending_store = [None, None, None, None]
        prev_ld = [None, None]
        prev_rdmas = [None, None]

        def process(p, sp):
            for rd in prev_rdmas[p]:
                rd.wait()
            for r in pairs[p]:
                if pending_store[r.idx] is not None:
                    pending_store[r.idx].wait()
                    pending_store[r.idx] = None
                if sp <= N_STEP - 2:
                    pl.semaphore_signal(
                        r.credit, inc=1, device_id=(r.credit_to,),
                        device_id_type=pl.DeviceIdType.MESH,
                    )
            if sp <= N_DEV - 2:
                for i, r in enumerate(pairs[p]):
                    prev_ld[i].wait()
                    add_tmp(r, (sp + 1) % 2)
                if sp == N_DEV - 2:
                    for r in pairs[p]:
                        relu_slot(r, 1)
                        pending_store[r.idx] = start_store(
                            r, 1, own_chunk_idx(r))
            else:
                t = sp - (N_DEV - 1)
                for r in pairs[p]:
                    pending_store[r.idx] = start_store(
                        r, (sp + 1) % 2, ag_chunk_idx(r, t))

        for k in range(2 * N_STEP):
            p = k % 2
            s = k // 2
            pair = pairs[p]

            if s >= 1:
                for r in pair:
                    pl.semaphore_wait(r.credit, 1)

            rdmas = []
            for r in pair:
                rd = pltpu.make_async_remote_copy(
                    src_ref=r.acc.at[s % 2],
                    dst_ref=r.acc.at[(s + 1) % 2],
                    send_sem=r.snd.at[s % 2],
                    recv_sem=r.rcv.at[(s + 1) % 2],
                    device_id=(r.dst,),
                    device_id_type=pl.DeviceIdType.MESH,
                )
                rd.start()
                rdmas.append(rd)
            prev_rdmas[p] = rdmas

            if k >= 1:
                process(1 - p, (k - 1) // 2)

            if s <= N_DEV - 2:
                for i, r in enumerate(pair):
                    prev_ld[i] = start_load(r, add_chunk_idx(r, s))

        process(1, N_STEP - 1)
        for d in pending_store:
            if d is not None:
                d.wait()

    return pl.pallas_call(
        body,
        out_shape=jax.ShapeDtypeStruct((M, N), jnp.float32),
        in_specs=[pl.BlockSpec(memory_space=pl.ANY)],
        out_specs=pl.BlockSpec(memory_space=pl.ANY),
        scratch_shapes=[
            pltpu.VMEM((2, CHUNK, Q), jnp.float32),
            pltpu.VMEM((2, CHUNK, Q), jnp.float32),
            pltpu.VMEM((2, CHUNK, Q), jnp.float32),
            pltpu.VMEM((2, CHUNK, Q), jnp.float32),
            pltpu.VMEM((CHUNK, Q), jnp.float32),
            pltpu.VMEM((CHUNK, Q), jnp.float32),
            pltpu.SemaphoreType.DMA((2,)),
            pltpu.SemaphoreType.DMA((2,)),
            pltpu.SemaphoreType.DMA((2,)),
            pltpu.SemaphoreType.DMA((2,)),
            pltpu.SemaphoreType.DMA((2,)),
            pltpu.SemaphoreType.DMA((2,)),
            pltpu.SemaphoreType.DMA((2,)),
            pltpu.SemaphoreType.DMA((2,)),
            pltpu.SemaphoreType.DMA,
            pltpu.SemaphoreType.DMA,
            pltpu.SemaphoreType.DMA((4,)),
            pltpu.SemaphoreType.REGULAR,
            pltpu.SemaphoreType.REGULAR,
            pltpu.SemaphoreType.REGULAR,
            pltpu.SemaphoreType.REGULAR,
        ],
        compiler_params=pltpu.CompilerParams(
            collective_id=0,
            vmem_limit_bytes=60 * 1024 * 1024,
        ),
    )(partial)


# device time: 1364260 ns/iter; 1.1072x vs baseline; 1.0652x over previous
import jax

try:
    jax.config.update("jax_compilation_cache_dir", "/tmp/scband_jax_cache")
    jax.config.update("jax_persistent_cache_min_compile_time_secs", 1.0)
except Exception:
    pass

import jax.numpy as jnp
from jax import lax
from jax.experimental import pallas as pl
from jax.experimental.pallas import tpu as pltpu

N_DEV = 8
N_STEP = 2 * (N_DEV - 1)


class _Ring:
    def __init__(self, idx, acc, snd, rcv, fwd, q):
        self.idx, self.acc, self.snd, self.rcv, self.fwd, self.q = (
            idx, acc, snd, rcv, fwd, q)


def kernel(x, w_mat):
    M, K = x.shape
    N = w_mat.shape[1]
    CHUNK = M // N_DEV
    Q = N // 4

    def body(x_ref, w_ref, out_ref,
             acc0, acc1, acc2, acc3, xb, wb_f, wb_r,
             snd0, rcv0, snd1, rcv1, snd2, rcv2, snd3, rcv3,
             ld_x_sem, ld_wf_sem, ld_wr_sem, store_sems,
             cr0, cr1, cr2, cr3):
        my = lax.axis_index("i")
        left = lax.rem(my - 1 + N_DEV, N_DEV)
        right = lax.rem(my + 1, N_DEV)

        barrier = pltpu.get_barrier_semaphore()
        for nbr in (left, right):
            pl.semaphore_signal(
                barrier, inc=1, device_id=(nbr,),
                device_id_type=pl.DeviceIdType.MESH,
            )
        pl.semaphore_wait(barrier, 2)

        rings = [
            _Ring(0, acc0, snd0, rcv0, True, 0),
            _Ring(1, acc1, snd1, rcv1, True, 1),
            _Ring(2, acc2, snd2, rcv2, False, 2),
            _Ring(3, acc3, snd3, rcv3, False, 3),
        ]
        credits = [cr0, cr1, cr2, cr3]
        for r in rings:
            r.dst = right if r.fwd else left
            r.credit_to = left if r.fwd else right
            r.wb = wb_f if r.fwd else wb_r
            r.wb_sem = ld_wf_sem if r.fwd else ld_wr_sem
            r.credit = credits[r.idx]
        pairs = [(rings[0], rings[2]), (rings[1], rings[3])]

        def rows(c):
            return pl.ds(c * CHUNK, CHUNK)

        def cols(q):
            return pl.ds(q * Q, Q)

        def mod8(v):
            return lax.rem(v + 2 * N_DEV, N_DEV)

        def add_chunk_idx(r, s):
            return mod8(my - s - 1) if r.fwd else mod8(my + s + 1)

        def own_chunk_idx(r):
            return mod8(my + 1) if r.fwd else mod8(my - 1)

        def ag_chunk_idx(r, t):
            return mod8(my - t) if r.fwd else mod8(my + t)

        def load_x(c):
            d = pltpu.make_async_copy(
                x_ref.at[rows(c), :], xb, ld_x_sem)
            d.start()
            return d

        def load_w(r):
            d = pltpu.make_async_copy(
                w_ref.at[:, cols(r.q)], r.wb, r.wb_sem)
            d.start()
            return d

        def mm(r, slot, accumulate):
            for kk in range(2):
                sub = pl.ds(kk * (CHUNK // 2), CHUNK // 2)
                prod = jnp.dot(xb[sub, :], r.wb[...],
                               preferred_element_type=jnp.float32)
                if accumulate:
                    r.acc[slot, sub] = r.acc[slot, sub] + prod
                else:
                    r.acc[slot, sub] = prod

        def relu_slot(r, slot):
            for kk in range(2):
                sub = pl.ds(kk * (CHUNK // 2), CHUNK // 2)
                r.acc[slot, sub] = jnp.maximum(r.acc[slot, sub], 0.0)

        def start_store(r, slot, c):
            d = pltpu.make_async_copy(
                r.acc.at[slot], out_ref.at[rows(c), cols(r.q)],
                store_sems.at[r.idx])
            d.start()
            return d

        def seed_pair(p, lx):
            lws = [load_w(r) for r in pairs[p]]
            if lx is not None:
                lx.wait()
            for r, lw in zip(pairs[p], lws):
                lw.wait()
                mm(r, 0, accumulate=False)

        pending_store = [None, None, None, None]
        prev_ld = [None, None, None]
        prev_rdmas = [None, None]

        def process(p, sp):
            for rd in prev_rdmas[p]:
                rd.wait()
            for r in pairs[p]:
                if pending_store[r.idx] is not None:
                    pending_store[r.idx].wait()
                    pending_store[r.idx] = None
                if sp <= N_STEP - 2:
                    pl.semaphore_signal(
                        r.credit, inc=1, device_id=(r.credit_to,),
                        device_id_type=pl.DeviceIdType.MESH,
                    )
            if sp <= N_DEV - 2:
                fr, rr = pairs[p]
                for d in prev_ld:
                    d.wait()
                mm(fr, (sp + 1) % 2, accumulate=True)
                lx = load_x(add_chunk_idx(rr, sp))
                lx.wait()
                mm(rr, (sp + 1) % 2, accumulate=True)
                if sp == N_DEV - 2:
                    for r in pairs[p]:
                        relu_slot(r, 1)
                        pending_store[r.idx] = start_store(
                            r, 1, own_chunk_idx(r))
            else:
                t = sp - (N_DEV - 1)
                for r in pairs[p]:
                    pending_store[r.idx] = start_store(
                        r, (sp + 1) % 2, ag_chunk_idx(r, t))

        seed_pair(0, load_x(my))

        for k in range(2 * N_STEP):
            p = k % 2
            s = k // 2
            pair = pairs[p]

            if s >= 1:
                for r in pair:
                    pl.semaphore_wait(r.credit, 1)

            rdmas = []
            for r in pair:
                rd = pltpu.make_async_remote_copy(
                    src_ref=r.acc.at[s % 2],
                    dst_ref=r.acc.at[(s + 1) % 2],
                    send_sem=r.snd.at[s % 2],
                    recv_sem=r.rcv.at[(s + 1) % 2],
                    device_id=(r.dst,),
                    device_id_type=pl.DeviceIdType.MESH,
                )
                rd.start()
                rdmas.append(rd)
            prev_rdmas[p] = rdmas

            if k == 0:
                seed_pair(1, None)
            else:
                process(1 - p, (k - 1) // 2)

            if s <= N_DEV - 2:
                fr, rr = pair
                prev_ld = [load_x(add_chunk_idx(fr, s)),
                           load_w(fr), load_w(rr)]

        process(1, N_STEP - 1)
        for d in pending_store:
            if d is not None:
                d.wait()

    return pl.pallas_call(
        body,
        out_shape=jax.ShapeDtypeStruct((M, N), jnp.float32),
        in_specs=[pl.BlockSpec(memory_space=pl.ANY),
                  pl.BlockSpec(memory_space=pl.ANY)],
        out_specs=pl.BlockSpec(memory_space=pl.ANY),
        scratch_shapes=[
            pltpu.VMEM((2, CHUNK, Q), jnp.float32),
            pltpu.VMEM((2, CHUNK, Q), jnp.float32),
            pltpu.VMEM((2, CHUNK, Q), jnp.float32),
            pltpu.VMEM((2, CHUNK, Q), jnp.float32),
            pltpu.VMEM((CHUNK, K), jnp.float32),
            pltpu.VMEM((K, Q), jnp.float32),
            pltpu.VMEM((K, Q), jnp.float32),
            pltpu.SemaphoreType.DMA((2,)),
            pltpu.SemaphoreType.DMA((2,)),
            pltpu.SemaphoreType.DMA((2,)),
            pltpu.SemaphoreType.DMA((2,)),
            pltpu.SemaphoreType.DMA((2,)),
            pltpu.SemaphoreType.DMA((2,)),
            pltpu.SemaphoreType.DMA((2,)),
            pltpu.SemaphoreType.DMA((2,)),
            pltpu.SemaphoreType.DMA,
            pltpu.SemaphoreType.DMA,
            pltpu.SemaphoreType.DMA,
            pltpu.SemaphoreType.DMA((4,)),
            pltpu.SemaphoreType.REGULAR,
            pltpu.SemaphoreType.REGULAR,
            pltpu.SemaphoreType.REGULAR,
            pltpu.SemaphoreType.REGULAR,
        ],
        compiler_params=pltpu.CompilerParams(
            collective_id=0,
            vmem_limit_bytes=60 * 1024 * 1024,
        ),
    )(x, w_mat)
